# baseline (speedup 1.0000x reference)
import functools

import jax
import jax.numpy as jnp
from jax import lax
from jax.experimental import pallas as pl
from jax.experimental.pallas import tpu as pltpu
from jax.experimental.pallas import tpu_sc as plsc

_D = 64
_DP = 128
_NW = 32
_CB = 256
_VB = 8192


def _transpose_body(in_ref, out_ref):
    out_ref[:, 0:_D] = in_ref[...].T


def _retile_out(flat2d, n, m):
    bk = 512
    g = m // 128

    def body(in_ref, out_ref):
        x = in_ref[...].reshape(bk, g, 128)
        out_ref[...] = jnp.transpose(x, (1, 2, 0)).reshape(m, bk)

    return pl.pallas_call(
        body,
        grid=(n // bk,),
        in_specs=[pl.BlockSpec((bk * g, 128), lambda i: (i, 0))],
        out_specs=pl.BlockSpec((m, bk), lambda i: (0, i)),
        out_shape=jax.ShapeDtypeStruct((m, n), jnp.float32),
    )(flat2d)


def _widen_table(emb_t):
    d, v = emb_t.shape
    grid = (v + _VB - 1) // _VB
    return pl.pallas_call(
        _transpose_body,
        grid=(grid,),
        in_specs=[pl.BlockSpec((d, _VB), lambda i: (0, i))],
        out_specs=pl.BlockSpec((_VB, _DP), lambda i: (i, 0)),
        out_shape=jax.ShapeDtypeStruct((v, _DP), jnp.float32),
    )(emb_t)


@jax.jit
def _gather_nn(emb_t, x_t):
    emb_pad = _widen_table(emb_t)
    h, b = x_t.shape
    bw = b // _NW
    nsub = bw // _CB
    n_chunks = h * nsub

    mesh = plsc.VectorSubcoreMesh(core_axis_name="c", subcore_axis_name="s")

    @functools.partial(
        pl.kernel,
        mesh=mesh,
        out_type=jax.ShapeDtypeStruct((b, h * _D), jnp.float32),
        scratch_types=[
            pltpu.VMEM((h, bw), jnp.int32),
            pltpu.VMEM((_CB, _DP), jnp.float32),
            pltpu.VMEM((_CB, _DP), jnp.float32),
            pltpu.SemaphoreType.DMA,
            pltpu.SemaphoreType.DMA,
            pltpu.SemaphoreType.DMA,
            pltpu.SemaphoreType.DMA,
        ],
        compiler_params=pltpu.CompilerParams(use_tc_tiling_on_sc=False),
    )
    def k(table_hbm, xt_hbm, out_hbm, idx_t, rows0, rows1,
          gsem0, gsem1, wsem0, wsem1):
        wid = lax.axis_index("s") * 2 + lax.axis_index("c")
        base = wid * bw
        rows = (rows0, rows1)
        gsem = (gsem0, gsem1)
        wsem = (wsem0, wsem1)

        pltpu.sync_copy(xt_hbm.at[:, pl.ds(base, bw)], idx_t)

        def gather(c):
            hh, half = c // nsub, c % nsub
            return pltpu.async_copy(
                table_hbm.at[idx_t.at[hh, pl.ds(half * _CB, _CB)]],
                rows[c % 2], gsem[c % 2])

        def writeback(c):
            hh, half = c // nsub, c % nsub
            return pltpu.async_copy(
                rows[c % 2].at[:, pl.ds(0, _D)],
                out_hbm.at[pl.ds(base + half * _CB, _CB),
                           pl.ds(hh * _D, _D)],
                wsem[c % 2])

        g_pending = gather(0)
        w_pending = [None, None]
        for c in range(n_chunks):
            s = c % 2
            g_pending.wait()
            if c + 1 < n_chunks:
                if w_pending[1 - s] is not None:
                    w_pending[1 - s].wait()
                g_pending = gather(c + 1)
            w_pending[s] = writeback(c)
        w_pending[(n_chunks - 2) % 2].wait()
        w_pending[(n_chunks - 1) % 2].wait()

    out = k(emb_pad, x_t)
    out_t = _retile_out(out.reshape(b * h * _D // 128, 128), b, h * _D)
    return jnp.transpose(out_t.reshape(h, _D, b), (2, 0, 1))


def kernel(x, embedding):
    return _gather_nn(jnp.swapaxes(embedding, 0, 1), jnp.swapaxes(x, 0, 1))


# speedup vs baseline: 1.8946x; 1.8946x over previous
import functools

import jax
import jax.numpy as jnp
from jax import lax
from jax.experimental import pallas as pl
from jax.experimental.pallas import tpu as pltpu
from jax.experimental.pallas import tpu_sc as plsc

_D = 64
_DP = 128
_NW = 32
_CB = 256
_VB = 16384


def _transpose_body(in_ref, out_ref):
    out_ref[:, 0:_D] = in_ref[...].T


def _retile_body(in_ref, out_ref):
    out_ref[...] = in_ref[...].T


def _retile_out(flat2d):
    n, m = flat2d.shape
    bk = 2048
    return pl.pallas_call(
        _retile_body,
        grid=(n // bk,),
        in_specs=[pl.BlockSpec((bk, m), lambda i: (i, 0))],
        out_specs=pl.BlockSpec((m, bk), lambda i: (0, i)),
        out_shape=jax.ShapeDtypeStruct((m, n), jnp.float32),
    )(flat2d)


def _widen_table(emb_t):
    d, v = emb_t.shape
    grid = (v + _VB - 1) // _VB
    return pl.pallas_call(
        _transpose_body,
        grid=(grid,),
        in_specs=[pl.BlockSpec((d, _VB), lambda i: (0, i))],
        out_specs=pl.BlockSpec((_VB, _DP), lambda i: (i, 0)),
        out_shape=jax.ShapeDtypeStruct((v, _DP), jnp.float32),
    )(emb_t)


@jax.jit
def _gather_nn(emb_t, x_t):
    emb_pad = _widen_table(emb_t)
    h, b = x_t.shape
    bw = b // _NW
    nsub = bw // _CB
    n_chunks = h * nsub

    mesh = plsc.VectorSubcoreMesh(core_axis_name="c", subcore_axis_name="s")

    @functools.partial(
        pl.kernel,
        mesh=mesh,
        out_type=jax.ShapeDtypeStruct((b, h * _D), jnp.float32),
        scratch_types=[
            pltpu.VMEM((h, bw), jnp.int32),
            pltpu.VMEM((_CB, _DP), jnp.float32),
            pltpu.VMEM((_CB, _DP), jnp.float32),
            pltpu.SemaphoreType.DMA,
            pltpu.SemaphoreType.DMA,
            pltpu.SemaphoreType.DMA,
            pltpu.SemaphoreType.DMA,
        ],
        compiler_params=pltpu.CompilerParams(use_tc_tiling_on_sc=False),
    )
    def k(table_hbm, xt_hbm, out_hbm, idx_t, rows0, rows1,
          gsem0, gsem1, wsem0, wsem1):
        wid = lax.axis_index("s") * 2 + lax.axis_index("c")
        base = wid * bw
        rows = (rows0, rows1)
        gsem = (gsem0, gsem1)
        wsem = (wsem0, wsem1)

        pltpu.sync_copy(xt_hbm.at[:, pl.ds(base, bw)], idx_t)

        def gather(c):
            hh, half = c // nsub, c % nsub
            return pltpu.async_copy(
                table_hbm.at[idx_t.at[hh, pl.ds(half * _CB, _CB)]],
                rows[c % 2], gsem[c % 2])

        def writeback(c):
            hh, half = c // nsub, c % nsub
            return pltpu.async_copy(
                rows[c % 2].at[:, pl.ds(0, _D)],
                out_hbm.at[pl.ds(base + half * _CB, _CB),
                           pl.ds(hh * _D, _D)],
                wsem[c % 2])

        g_pending = gather(0)
        w_pending = [None, None]
        for c in range(n_chunks):
            s = c % 2
            g_pending.wait()
            if c + 1 < n_chunks:
                if w_pending[1 - s] is not None:
                    w_pending[1 - s].wait()
                g_pending = gather(c + 1)
            w_pending[s] = writeback(c)
        w_pending[(n_chunks - 2) % 2].wait()
        w_pending[(n_chunks - 1) % 2].wait()

    out = k(emb_pad, x_t)
    out_t = _retile_out(out)
    return jnp.transpose(out_t.reshape(h, _D, b), (2, 0, 1))


def kernel(x, embedding):
    return _gather_nn(jnp.swapaxes(embedding, 0, 1), jnp.swapaxes(x, 0, 1))


# speedup vs baseline: 1.9087x; 1.0074x over previous
import functools

import jax
import jax.numpy as jnp
from jax import lax
from jax.experimental import pallas as pl
from jax.experimental.pallas import tpu as pltpu
from jax.experimental.pallas import tpu_sc as plsc

_D = 64
_DP = 128
_NW = 32
_CB = 256
_VB = 32768


def _transpose_body(in_ref, out_ref):
    out_ref[:, 0:_D] = in_ref[...].T


def _retile_body(in_ref, out_ref):
    out_ref[...] = in_ref[...].T


def _retile_out(flat2d):
    n, m = flat2d.shape
    bk = 2048
    return pl.pallas_call(
        _retile_body,
        grid=(n // bk,),
        in_specs=[pl.BlockSpec((bk, m), lambda i: (i, 0))],
        out_specs=pl.BlockSpec((m, bk), lambda i: (0, i)),
        out_shape=jax.ShapeDtypeStruct((m, n), jnp.float32),
    )(flat2d)


def _widen_table(emb_t):
    d, v = emb_t.shape
    grid = (v + _VB - 1) // _VB
    return pl.pallas_call(
        _transpose_body,
        grid=(grid,),
        in_specs=[pl.BlockSpec((d, _VB), lambda i: (0, i))],
        out_specs=pl.BlockSpec((_VB, _DP), lambda i: (i, 0)),
        out_shape=jax.ShapeDtypeStruct((v, _DP), jnp.float32),
    )(emb_t)


@jax.jit
def _gather_nn(emb_t, x_t):
    emb_pad = _widen_table(emb_t)
    h, b = x_t.shape
    bw = b // _NW
    nsub = bw // _CB
    n_chunks = h * nsub

    mesh = plsc.VectorSubcoreMesh(core_axis_name="c", subcore_axis_name="s")

    @functools.partial(
        pl.kernel,
        mesh=mesh,
        out_type=jax.ShapeDtypeStruct((b, h * _D), jnp.float32),
        scratch_types=[
            pltpu.VMEM((h, bw), jnp.int32),
            pltpu.VMEM((_CB, _DP), jnp.float32),
            pltpu.VMEM((_CB, _DP), jnp.float32),
            pltpu.SemaphoreType.DMA,
            pltpu.SemaphoreType.DMA,
            pltpu.SemaphoreType.DMA,
            pltpu.SemaphoreType.DMA,
        ],
        compiler_params=pltpu.CompilerParams(use_tc_tiling_on_sc=False),
    )
    def k(table_hbm, xt_hbm, out_hbm, idx_t, rows0, rows1,
          gsem0, gsem1, wsem0, wsem1):
        wid = lax.axis_index("s") * 2 + lax.axis_index("c")
        base = wid * bw
        rows = (rows0, rows1)
        gsem = (gsem0, gsem1)
        wsem = (wsem0, wsem1)

        pltpu.sync_copy(xt_hbm.at[:, pl.ds(base, bw)], idx_t)

        def gather(c):
            hh, half = c // nsub, c % nsub
            return pltpu.async_copy(
                table_hbm.at[idx_t.at[hh, pl.ds(half * _CB, _CB)]],
                rows[c % 2], gsem[c % 2])

        def writeback(c):
            hh, half = c // nsub, c % nsub
            return pltpu.async_copy(
                rows[c % 2].at[:, pl.ds(0, _D)],
                out_hbm.at[pl.ds(base + half * _CB, _CB),
                           pl.ds(hh * _D, _D)],
                wsem[c % 2])

        g_pending = gather(0)
        w_pending = [None, None]
        for c in range(n_chunks):
            s = c % 2
            g_pending.wait()
            if c + 1 < n_chunks:
                if w_pending[1 - s] is not None:
                    w_pending[1 - s].wait()
                g_pending = gather(c + 1)
            w_pending[s] = writeback(c)
        w_pending[(n_chunks - 2) % 2].wait()
        w_pending[(n_chunks - 1) % 2].wait()

    out = k(emb_pad, x_t)
    out_t = _retile_out(out)
    return jnp.transpose(out_t.reshape(h, _D, b), (2, 0, 1))


def kernel(x, embedding):
    return _gather_nn(jnp.swapaxes(embedding, 0, 1), jnp.swapaxes(x, 0, 1))


# speedup vs baseline: 1.9755x; 1.0350x over previous
import functools

import jax
import jax.numpy as jnp
from jax import lax
from jax.experimental import pallas as pl
from jax.experimental.pallas import tpu as pltpu
from jax.experimental.pallas import tpu_sc as plsc

_D = 64
_DP = 128
_NW = 32
_CB = 256
_VB = 32768


def _transpose_body(in_ref, out_ref):
    out_ref[:, 0:_D] = in_ref[...].T


def _retile_body(in_ref, out_ref):
    out_ref[...] = in_ref[...].T


def _retile_out(flat2d):
    n, m = flat2d.shape
    bk = 2048
    return pl.pallas_call(
        _retile_body,
        grid=(n // bk,),
        in_specs=[pl.BlockSpec((bk, m), lambda i: (i, 0))],
        out_specs=pl.BlockSpec((m, bk), lambda i: (0, i)),
        out_shape=jax.ShapeDtypeStruct((m, n), jnp.float32),
    )(flat2d)


def _widen_table(emb_t):
    d, v = emb_t.shape
    grid = (v + _VB - 1) // _VB
    return pl.pallas_call(
        _transpose_body,
        grid=(grid,),
        in_specs=[pl.BlockSpec((d, _VB), lambda i: (0, i))],
        out_specs=pl.BlockSpec((_VB, _DP), lambda i: (i, 0)),
        out_shape=jax.ShapeDtypeStruct((v, _DP), jnp.float32),
    )(emb_t)


@jax.jit
def _gather_nn(emb_t, x_t):
    emb_pad = _widen_table(emb_t)
    h, b = x_t.shape
    bw = b // _NW
    nsub = bw // _CB
    n_chunks = h * nsub

    mesh = plsc.VectorSubcoreMesh(core_axis_name="c", subcore_axis_name="s")

    @functools.partial(
        pl.kernel,
        mesh=mesh,
        out_type=jax.ShapeDtypeStruct((b, h * _D), jnp.float32),
        scratch_types=[
            pltpu.VMEM((h, bw), jnp.int32),
            pltpu.VMEM((_CB, _DP), jnp.float32),
            pltpu.VMEM((_CB, _DP), jnp.float32),
            pltpu.VMEM((_CB, _DP), jnp.float32),
            pltpu.SemaphoreType.DMA,
            pltpu.SemaphoreType.DMA,
            pltpu.SemaphoreType.DMA,
            pltpu.SemaphoreType.DMA,
            pltpu.SemaphoreType.DMA,
            pltpu.SemaphoreType.DMA,
        ],
        compiler_params=pltpu.CompilerParams(use_tc_tiling_on_sc=False),
    )
    def k(table_hbm, xt_hbm, out_hbm, idx_t, rows0, rows1, rows2,
          gsem0, gsem1, gsem2, wsem0, wsem1, wsem2):
        wid = lax.axis_index("s") * 2 + lax.axis_index("c")
        base = wid * bw
        rows = (rows0, rows1, rows2)
        gsem = (gsem0, gsem1, gsem2)
        wsem = (wsem0, wsem1, wsem2)

        pltpu.sync_copy(xt_hbm.at[:, pl.ds(base, bw)], idx_t)

        nb = 3

        def gather(c):
            hh, half = c // nsub, c % nsub
            return pltpu.async_copy(
                table_hbm.at[idx_t.at[hh, pl.ds(half * _CB, _CB)]],
                rows[c % nb], gsem[c % nb])

        def writeback(c):
            hh, half = c // nsub, c % nsub
            return pltpu.async_copy(
                rows[c % nb].at[:, pl.ds(0, _D)],
                out_hbm.at[pl.ds(base + half * _CB, _CB),
                           pl.ds(hh * _D, _D)],
                wsem[c % nb])

        g_pending = {c: gather(c) for c in range(nb - 1)}
        w_pending = [None] * nb
        for c in range(n_chunks):
            s = c % nb
            g_pending.pop(c).wait()
            nxt = c + nb - 1
            if nxt < n_chunks:
                sn = nxt % nb
                if w_pending[sn] is not None:
                    w_pending[sn].wait()
                    w_pending[sn] = None
                g_pending[nxt] = gather(nxt)
            w_pending[s] = writeback(c)
        for s in range(nb):
            if w_pending[s] is not None:
                w_pending[s].wait()

    out = k(emb_pad, x_t)
    out_t = _retile_out(out)
    return jnp.transpose(out_t.reshape(h, _D, b), (2, 0, 1))


def kernel(x, embedding):
    return _gather_nn(jnp.swapaxes(embedding, 0, 1), jnp.swapaxes(x, 0, 1))
